# baseline (speedup 1.0000x reference)
import functools

import jax
import jax.numpy as jnp
from jax import lax
from jax.experimental import pallas as pl
from jax.experimental.pallas import tpu as pltpu
from jax.experimental.pallas import tpu_sc as plsc

VOCAB = 1000000
CONTEXT = 20
EMBED = 64
BATCH = 16384

NC = 2
NS = 16
NW = NC * NS
ROWS_W = BATCH // NW
CHUNK_B = 4
CHUNK_R = CHUNK_B * CONTEXT
NCHUNK = ROWS_W // CHUNK_B

VP = VOCAB // 2
BLK = 4096
GRID_T = (VP + BLK - 1) // BLK


def _ctx_sums(inputs, emb_table):
  mesh = plsc.VectorSubcoreMesh(core_axis_name="c", subcore_axis_name="s")

  @functools.partial(
      pl.kernel,
      mesh=mesh,
      out_type=jax.ShapeDtypeStruct((NC, CONTEXT, EMBED), jnp.float32),
      compiler_params=pltpu.CompilerParams(use_tc_tiling_on_sc=False),
      scratch_types=[
          pltpu.VMEM((ROWS_W, CONTEXT), jnp.int32),
          pltpu.VMEM((ROWS_W * CONTEXT,), jnp.int32),
          pltpu.VMEM((CHUNK_R,), jnp.int32),
          pltpu.VMEM((CHUNK_R, EMBED), jnp.float32),
          pltpu.VMEM((CHUNK_R, EMBED), jnp.float32),
          pltpu.VMEM((CONTEXT, EMBED), jnp.float32),
          pltpu.VMEM_SHARED((CONTEXT, EMBED), jnp.float32),
          pltpu.SemaphoreType.DMA,
          pltpu.SemaphoreType.DMA,
      ],
  )
  def k(idx_hbm, emb_hbm, out_hbm,
        idx_v2, idx_v, lidx_v, buf_a, buf_b, zero_v, acc_sh, sem_a, sem_b):
    c = lax.axis_index("c")
    s = lax.axis_index("s")
    wid = s * NC + c
    pltpu.sync_copy(idx_hbm.at[pl.ds(wid * ROWS_W, ROWS_W), :], idx_v2)
    def flat_body(r, carry):
      v0 = idx_v2[r, pl.ds(0, 16)]
      v1 = idx_v2[r, pl.ds(CONTEXT - 16, 16)]
      idx_v[pl.ds(r * CONTEXT, 16)] = v0
      idx_v[pl.ds(r * CONTEXT + CONTEXT - 16, 16)] = v1
      return carry

    lax.fori_loop(0, ROWS_W, flat_body, 0)
    for q in range(CHUNK_R // 16):
      lidx_v[pl.ds(q * 16, 16)] = lax.rem(
          lax.iota(jnp.int32, 16) + q * 16, CONTEXT)
    for l in range(CONTEXT):
      for q in range(EMBED // 16):
        zero_v[l, pl.ds(q * 16, 16)] = jnp.zeros((16,), jnp.float32)

    @pl.when(s == 0)
    def _():
      pltpu.sync_copy(zero_v, acc_sh)

    plsc.subcore_barrier()

    def gather(kk, buf, sem):
      return pltpu.make_async_copy(
          emb_hbm.at[idx_v.at[pl.ds(kk * CHUNK_R, CHUNK_R)]], buf, sem)

    gather(0, buf_a, sem_a).start()

    def body(i, carry):
      k0 = 2 * i
      gather(k0 + 1, buf_b, sem_b).start()
      gather(k0, buf_a, sem_a).wait()
      pltpu.sync_copy(buf_a, acc_sh.at[lidx_v], add=True)

      @pl.when(i + 1 < NCHUNK // 2)
      def _():
        gather(k0 + 2, buf_a, sem_a).start()

      gather(k0 + 1, buf_b, sem_b).wait()
      pltpu.sync_copy(buf_b, acc_sh.at[lidx_v], add=True)
      return carry

    lax.fori_loop(0, NCHUNK // 2, body, 0)

    plsc.subcore_barrier()

    @pl.when(s == 0)
    def _():
      pltpu.sync_copy(acc_sh, out_hbm.at[c])

  return k(inputs, emb_table)


def _pass_a(partials, w2, be, bo):
  def body(p_ref, w_ref, be_ref, bo_ref, out_ref, st_ref, h_s, m_s, s_s):
    i = pl.program_id(0)

    @pl.when(i == 0)
    def _():
      h = jnp.tanh((p_ref[0] + p_ref[1]) * (1.0 / BATCH))
      z = jnp.zeros((CONTEXT, EMBED), jnp.float32)
      h_s[...] = jnp.concatenate(
          [jnp.concatenate([h, z], axis=1),
           jnp.concatenate([z, h], axis=1)], axis=0)

    logits = lax.dot_general(h_s[...], w_ref[...], (((1,), (1,)), ((), ())),
                             preferred_element_type=jnp.float32)
    row = lax.broadcasted_iota(jnp.int32, (2 * CONTEXT, BLK), 0)
    bias = jnp.where(row < CONTEXT,
                     jnp.broadcast_to(be_ref[...], (2 * CONTEXT, BLK)),
                     jnp.broadcast_to(bo_ref[...], (2 * CONTEXT, BLK)))
    logits = logits + bias
    col = i * BLK + lax.broadcasted_iota(jnp.int32, (2 * CONTEXT, BLK), 1)
    logits = jnp.where(col < VP, logits, -jnp.inf)
    out_ref[...] = logits
    tmax = jnp.max(logits, axis=1, keepdims=True)

    @pl.when(i == 0)
    def _():
      m_s[...] = tmax
      s_s[...] = jnp.sum(jnp.exp(logits - tmax), axis=1, keepdims=True)

    @pl.when(i > 0)
    def _():
      m_old = m_s[...]
      m_new = jnp.maximum(m_old, tmax)
      s_s[...] = s_s[...] * jnp.exp(m_old - m_new) + jnp.sum(
          jnp.exp(logits - m_new), axis=1, keepdims=True)
      m_s[...] = m_new

    m_e = m_s[0:CONTEXT, :]
    m_o = m_s[CONTEXT:2 * CONTEXT, :]
    s_e = s_s[0:CONTEXT, :]
    s_o = s_s[CONTEXT:2 * CONTEXT, :]
    m_c = jnp.maximum(m_e, m_o)
    s_c = s_e * jnp.exp(m_e - m_c) + s_o * jnp.exp(m_o - m_c)
    st_ref[...] = m_c + jnp.log(s_c)

  return pl.pallas_call(
      body,
      grid=(GRID_T,),
      in_specs=[
          pl.BlockSpec((NC, CONTEXT, EMBED), lambda i: (0, 0, 0)),
          pl.BlockSpec((BLK, 2 * EMBED), lambda i: (i, 0)),
          pl.BlockSpec((1, BLK), lambda i: (0, i)),
          pl.BlockSpec((1, BLK), lambda i: (0, i)),
      ],
      out_specs=[
          pl.BlockSpec((2 * CONTEXT, BLK), lambda i: (0, i)),
          pl.BlockSpec((CONTEXT, 1), lambda i: (0, 0)),
      ],
      out_shape=[
          jax.ShapeDtypeStruct((2 * CONTEXT, VP), jnp.float32),
          jax.ShapeDtypeStruct((CONTEXT, 1), jnp.float32),
      ],
      scratch_shapes=[
          pltpu.VMEM((2 * CONTEXT, 2 * EMBED), jnp.float32),
          pltpu.VMEM((2 * CONTEXT, 1), jnp.float32),
          pltpu.VMEM((2 * CONTEXT, 1), jnp.float32),
      ],
      compiler_params=pltpu.CompilerParams(
          dimension_semantics=("arbitrary",)),
  )(partials, w2, be, bo)


def _pass_b(logits2, stats):
  def body(l_ref, st_ref, o_ref):
    x = l_ref[...] - jnp.concatenate([st_ref[...], st_ref[...]], axis=0)
    ev = x[0:CONTEXT, :]
    od = x[CONTEXT:2 * CONTEXT, :]
    o_ref[...] = jnp.stack([ev, od], axis=2).reshape(CONTEXT, 2 * BLK)

  return pl.pallas_call(
      body,
      grid=(GRID_T,),
      in_specs=[
          pl.BlockSpec((2 * CONTEXT, BLK), lambda i: (0, i)),
          pl.BlockSpec((CONTEXT, 1), lambda i: (0, 0)),
      ],
      out_specs=pl.BlockSpec((CONTEXT, 2 * BLK), lambda i: (0, i)),
      out_shape=jax.ShapeDtypeStruct((CONTEXT, VOCAB), jnp.float32),
  )(logits2, stats)


def kernel(inputs, emb_table, lin_w, lin_b):
  idx = inputs.astype(jnp.int32)
  partials = _ctx_sums(idx, emb_table)
  w2 = lin_w.reshape(VP, 2 * EMBED)
  be = lin_b[0::2].reshape(1, VP)
  bo = lin_b[1::2].reshape(1, VP)
  logits2, stats = _pass_a(partials, w2, be, bo)
  return _pass_b(logits2, stats)


# speedup vs baseline: 4.7285x; 4.7285x over previous
import functools

import jax
import jax.numpy as jnp
from jax import lax
from jax.experimental import pallas as pl
from jax.experimental.pallas import tpu as pltpu
from jax.experimental.pallas import tpu_sc as plsc

VOCAB = 1000000
CONTEXT = 20
EMBED = 64
BATCH = 16384

NC = 2
NS = 16
NW = NC * NS
ROWS_W = BATCH // NW
CHUNK_B = 4
CHUNK_R = CHUNK_B * CONTEXT
NCHUNK = ROWS_W // CHUNK_B

VT = 8192
GRID_T = (VOCAB + VT - 1) // VT


def _ctx_sums(inputs, emb_table):
  mesh = plsc.VectorSubcoreMesh(core_axis_name="c", subcore_axis_name="s")

  @functools.partial(
      pl.kernel,
      mesh=mesh,
      out_type=jax.ShapeDtypeStruct((NC, CONTEXT, EMBED), jnp.float32),
      compiler_params=pltpu.CompilerParams(use_tc_tiling_on_sc=False),
      scratch_types=[
          pltpu.VMEM((ROWS_W, CONTEXT), jnp.int32),
          pltpu.VMEM((ROWS_W * CONTEXT,), jnp.int32),
          pltpu.VMEM((CHUNK_R,), jnp.int32),
          pltpu.VMEM((CHUNK_R, EMBED), jnp.float32),
          pltpu.VMEM((CHUNK_R, EMBED), jnp.float32),
          pltpu.VMEM((CONTEXT, EMBED), jnp.float32),
          pltpu.VMEM_SHARED((CONTEXT, EMBED), jnp.float32),
          pltpu.SemaphoreType.DMA,
          pltpu.SemaphoreType.DMA,
      ],
  )
  def k(idx_hbm, emb_hbm, out_hbm,
        idx_v2, idx_v, lidx_v, buf_a, buf_b, zero_v, acc_sh, sem_a, sem_b):
    c = lax.axis_index("c")
    s = lax.axis_index("s")
    wid = s * NC + c
    pltpu.sync_copy(idx_hbm.at[pl.ds(wid * ROWS_W, ROWS_W), :], idx_v2)
    def flat_body(r, carry):
      v0 = idx_v2[r, pl.ds(0, 16)]
      v1 = idx_v2[r, pl.ds(CONTEXT - 16, 16)]
      idx_v[pl.ds(r * CONTEXT, 16)] = v0
      idx_v[pl.ds(r * CONTEXT + CONTEXT - 16, 16)] = v1
      return carry

    lax.fori_loop(0, ROWS_W, flat_body, 0)
    for q in range(CHUNK_R // 16):
      lidx_v[pl.ds(q * 16, 16)] = lax.rem(
          lax.iota(jnp.int32, 16) + q * 16, CONTEXT)
    for l in range(CONTEXT):
      for q in range(EMBED // 16):
        zero_v[l, pl.ds(q * 16, 16)] = jnp.zeros((16,), jnp.float32)

    @pl.when(s == 0)
    def _():
      pltpu.sync_copy(zero_v, acc_sh)

    plsc.subcore_barrier()

    def gather(kk, buf, sem):
      return pltpu.make_async_copy(
          emb_hbm.at[idx_v.at[pl.ds(kk * CHUNK_R, CHUNK_R)]], buf, sem)

    gather(0, buf_a, sem_a).start()

    def body(i, carry):
      k0 = 2 * i
      gather(k0 + 1, buf_b, sem_b).start()
      gather(k0, buf_a, sem_a).wait()
      pltpu.sync_copy(buf_a, acc_sh.at[lidx_v], add=True)

      @pl.when(i + 1 < NCHUNK // 2)
      def _():
        gather(k0 + 2, buf_a, sem_a).start()

      gather(k0 + 1, buf_b, sem_b).wait()
      pltpu.sync_copy(buf_b, acc_sh.at[lidx_v], add=True)
      return carry

    lax.fori_loop(0, NCHUNK // 2, body, 0)

    plsc.subcore_barrier()

    @pl.when(s == 0)
    def _():
      pltpu.sync_copy(acc_sh, out_hbm.at[c])

  return k(inputs, emb_table)


def _pass_a(partials, lin_w, lin_b2):
  def body(p_ref, w_ref, b_ref, out_ref, st_ref, h_s, m_s, s_s):
    i = pl.program_id(0)

    @pl.when(i == 0)
    def _():
      h_s[...] = jnp.tanh((p_ref[0] + p_ref[1]) * (1.0 / BATCH))

    logits = lax.dot_general(h_s[...], w_ref[...], (((1,), (1,)), ((), ())),
                             preferred_element_type=jnp.float32)
    logits = logits + b_ref[...]
    col = i * VT + lax.broadcasted_iota(jnp.int32, (CONTEXT, VT), 1)
    logits = jnp.where(col < VOCAB, logits, -jnp.inf)
    out_ref[...] = logits
    tmax = jnp.max(logits, axis=1, keepdims=True)

    @pl.when(i == 0)
    def _():
      m_s[...] = tmax
      s_s[...] = jnp.sum(jnp.exp(logits - tmax), axis=1, keepdims=True)

    @pl.when(i > 0)
    def _():
      m_old = m_s[...]
      m_new = jnp.maximum(m_old, tmax)
      s_s[...] = s_s[...] * jnp.exp(m_old - m_new) + jnp.sum(
          jnp.exp(logits - m_new), axis=1, keepdims=True)
      m_s[...] = m_new

    st_ref[...] = m_s[...] + jnp.log(s_s[...])

  return pl.pallas_call(
      body,
      grid=(GRID_T,),
      in_specs=[
          pl.BlockSpec((NC, CONTEXT, EMBED), lambda i: (0, 0, 0)),
          pl.BlockSpec((VT, EMBED), lambda i: (i, 0)),
          pl.BlockSpec((1, VT), lambda i: (0, i)),
      ],
      out_specs=[
          pl.BlockSpec((CONTEXT, VT), lambda i: (0, i)),
          pl.BlockSpec((CONTEXT, 1), lambda i: (0, 0)),
      ],
      out_shape=[
          jax.ShapeDtypeStruct((CONTEXT, VOCAB), jnp.float32),
          jax.ShapeDtypeStruct((CONTEXT, 1), jnp.float32),
      ],
      scratch_shapes=[
          pltpu.VMEM((CONTEXT, EMBED), jnp.float32),
          pltpu.VMEM((CONTEXT, 1), jnp.float32),
          pltpu.VMEM((CONTEXT, 1), jnp.float32),
      ],
      compiler_params=pltpu.CompilerParams(
          dimension_semantics=("arbitrary",)),
  )(partials, lin_w, lin_b2)


def _pass_b(logits, stats):
  def body(l_ref, st_ref, o_ref):
    o_ref[...] = l_ref[...] - st_ref[...]

  return pl.pallas_call(
      body,
      grid=(GRID_T,),
      in_specs=[
          pl.BlockSpec((CONTEXT, VT), lambda i: (0, i)),
          pl.BlockSpec((CONTEXT, 1), lambda i: (0, 0)),
      ],
      out_specs=pl.BlockSpec((CONTEXT, VT), lambda i: (0, i)),
      out_shape=jax.ShapeDtypeStruct((CONTEXT, VOCAB), jnp.float32),
      input_output_aliases={0: 0},
  )(logits, stats)


def kernel(inputs, emb_table, lin_w, lin_b):
  idx = inputs.astype(jnp.int32)
  partials = _ctx_sums(idx, emb_table)
  logits, stats = _pass_a(partials, lin_w, lin_b.reshape(1, VOCAB))
  return _pass_b(logits, stats)
